# baseline (speedup 1.0000x reference)
import functools

import jax
import jax.numpy as jnp
from jax import lax
from jax.experimental import pallas as pl
from jax.experimental.pallas import tpu as pltpu
from jax.experimental.pallas import tpu_sc as plsc

N = 10000
E = 320000
IN_FEAT = 128
HIDDEN = 64
OUT_FEAT = 64
GRID = 4

NC = 2
NS = 16
NTILES = NC * NS
BATCH = 128
NB = 80
NBUF = 4
E_PAD = NTILES * NB * BATCH
NP = 10240

ROW_BLK = 2000


def _lin_in_body(x_ref, w_ref, b_ref, o_ref):
    acc = lax.dot_general(
        x_ref[...], w_ref[...], (((1,), (1,)), ((), ())),
        preferred_element_type=jnp.float32)
    o_ref[...] = acc + b_ref[...]


def _lin_in(x, W_in, b_in):
    return pl.pallas_call(
        _lin_in_body,
        grid=(N // ROW_BLK,),
        in_specs=[
            pl.BlockSpec((ROW_BLK, IN_FEAT), lambda i: (i, 0)),
            pl.BlockSpec((HIDDEN, IN_FEAT), lambda i: (0, 0)),
            pl.BlockSpec((1, HIDDEN), lambda i: (0, 0)),
        ],
        out_specs=pl.BlockSpec((ROW_BLK, HIDDEN), lambda i: (i, 0)),
        out_shape=jax.ShapeDtypeStruct((N, HIDDEN), jnp.float32),
    )(x, W_in, b_in.reshape(1, HIDDEN))


def _sc_spmm_body(rows_hbm, cols_hbm, h_hbm, z_hbm, out_hbm,
                  colv, rowv, rbufs, acc, semg, sems):
    c = lax.axis_index("c")
    s = lax.axis_index("s")
    wid = c * NS + s
    rpt = NP // NS

    pltpu.sync_copy(z_hbm, acc.at[pl.ds(s * rpt, rpt)])
    pltpu.sync_copy(cols_hbm.at[wid], colv)
    pltpu.sync_copy(rows_hbm.at[wid], rowv)
    plsc.subcore_barrier()

    pltpu.async_copy(h_hbm.at[colv.at[0]], rbufs[0], semg[0])

    def body(j2, carry):
        j = j2 * 2
        jn = jnp.minimum(j + 2, NB - 1)
        pltpu.make_async_copy(h_hbm.at[colv.at[j]], rbufs[0], semg[0]).wait()
        pltpu.async_copy(h_hbm.at[colv.at[j + 1]], rbufs[1], semg[1])
        pltpu.sync_copy(rbufs[0], acc.at[rowv.at[j]], add=True)
        pltpu.make_async_copy(h_hbm.at[colv.at[j + 1]], rbufs[1],
                              semg[1]).wait()
        pltpu.async_copy(h_hbm.at[colv.at[jn]], rbufs[0], semg[0])
        pltpu.sync_copy(rbufs[1], acc.at[rowv.at[j + 1]], add=True)
        return carry

    lax.fori_loop(0, NB // 2, body, 0)
    pltpu.make_async_copy(h_hbm.at[colv.at[NB - 1]], rbufs[0],
                          semg[0]).wait()
    plsc.subcore_barrier()

    pltpu.sync_copy(acc.at[pl.ds(s * rpt, rpt)],
                    out_hbm.at[c, pl.ds(s * rpt, rpt)])


_sc_spmm = functools.partial(
    pl.kernel,
    out_type=jax.ShapeDtypeStruct((NC, NP, HIDDEN), jnp.float32),
    mesh=plsc.VectorSubcoreMesh(
        core_axis_name="c", subcore_axis_name="s",
        num_cores=NC, num_subcores=NS),
    scratch_types=[
        pltpu.VMEM((NB, BATCH), jnp.int32),
        pltpu.VMEM((NB, BATCH), jnp.int32),
        [pltpu.VMEM((BATCH, HIDDEN), jnp.float32) for _ in range(2)],
        pltpu.VMEM_SHARED((NP, HIDDEN), jnp.float32),
        [pltpu.SemaphoreType.DMA for _ in range(2)],
        [pltpu.SemaphoreType.DMA for _ in range(2)],
    ],
    compiler_params=pltpu.CompilerParams(use_tc_tiling_on_sc=False),
)(_sc_spmm_body)


def _post_body(p0_ref, p1_ref, wf_ref, wo_ref, o_ref):
    a = p0_ref[...] + p1_ref[...]
    feats = []
    for g in range(GRID):
        feats.append(jnp.cos((g + 1.0) * a))
    for g in range(GRID):
        feats.append(jnp.sin((g + 1.0) * a))
    feat = jnp.concatenate(feats, axis=1)
    y = jnp.dot(feat, wf_ref[...], preferred_element_type=jnp.float32)
    o = lax.dot_general(
        y, wo_ref[...], (((1,), (1,)), ((), ())),
        preferred_element_type=jnp.float32)
    m = jnp.max(o, axis=-1, keepdims=True)
    ex = jnp.exp(o - m)
    o_ref[...] = (o - m) - jnp.log(jnp.sum(ex, axis=-1, keepdims=True))


def _post(p0, p1, WF, W_out):
    F = 2 * GRID * HIDDEN
    return pl.pallas_call(
        _post_body,
        grid=(N // ROW_BLK,),
        in_specs=[
            pl.BlockSpec((ROW_BLK, HIDDEN), lambda i: (i, 0)),
            pl.BlockSpec((ROW_BLK, HIDDEN), lambda i: (i, 0)),
            pl.BlockSpec((F, HIDDEN), lambda i: (0, 0)),
            pl.BlockSpec((OUT_FEAT, HIDDEN), lambda i: (0, 0)),
        ],
        out_specs=pl.BlockSpec((ROW_BLK, OUT_FEAT), lambda i: (i, 0)),
        out_shape=jax.ShapeDtypeStruct((N, OUT_FEAT), jnp.float32),
    )(p0, p1, WF, W_out)


def kernel(x, edge_index, W_in, b_in, coeffs0, W_out):
    h = _lin_in(x, W_in, b_in)

    row = edge_index[0]
    col = edge_index[1]
    pad = E_PAD - E
    rowp = jnp.concatenate(
        [row, jnp.full((pad,), N, jnp.int32)]).reshape(NTILES, NB, BATCH)
    colp = jnp.concatenate(
        [col, jnp.zeros((pad,), jnp.int32)]).reshape(NTILES, NB, BATCH)
    zeros = jnp.zeros((NP // NS, HIDDEN), jnp.float32)

    partials = _sc_spmm(rowp, colp, h, zeros)

    WFc = jnp.transpose(coeffs0[0], (2, 1, 0)).reshape(GRID * HIDDEN, HIDDEN)
    WFs = jnp.transpose(coeffs0[1], (2, 1, 0)).reshape(GRID * HIDDEN, HIDDEN)
    WF = jnp.concatenate([WFc, WFs], axis=0)

    return _post(partials[0, :N], partials[1, :N], WF, W_out)


# speedup vs baseline: 2.1162x; 2.1162x over previous
import functools

import jax
import jax.numpy as jnp
from jax import lax
from jax.experimental import pallas as pl
from jax.experimental.pallas import tpu as pltpu
from jax.experimental.pallas import tpu_sc as plsc

N = 10000
E = 320000
IN_FEAT = 128
HIDDEN = 64
OUT_FEAT = 64
GRID = 4

NC = 2
NS = 16
NTILES = NC * NS
BATCH = 128
NBAT = E // BATCH
NB_LO = NBAT // NTILES
NB_REM = NBAT - NB_LO * NTILES
NP = 10240

ROW_BLK = 2000


def _lin_in_body(x_ref, w_ref, b_ref, o_ref):
    acc = lax.dot_general(
        x_ref[...], w_ref[...], (((1,), (1,)), ((), ())),
        preferred_element_type=jnp.float32)
    o_ref[...] = acc + b_ref[...]


def _lin_in(x, W_in, b_in):
    return pl.pallas_call(
        _lin_in_body,
        grid=(N // ROW_BLK,),
        in_specs=[
            pl.BlockSpec((ROW_BLK, IN_FEAT), lambda i: (i, 0)),
            pl.BlockSpec((HIDDEN, IN_FEAT), lambda i: (0, 0)),
            pl.BlockSpec((1, HIDDEN), lambda i: (0, 0)),
        ],
        out_specs=pl.BlockSpec((ROW_BLK, HIDDEN), lambda i: (i, 0)),
        out_shape=jax.ShapeDtypeStruct((N, HIDDEN), jnp.float32),
    )(x, W_in, b_in.reshape(1, HIDDEN))


def _sc_spmm_body(e3_hbm, h_hbm, z_hbm, out_hbm,
                  colv, rowv, rbuf, acc, sem):
    c = lax.axis_index("c")
    s = lax.axis_index("s")
    wid = c * NS + s
    rpt = NP // NS

    pltpu.sync_copy(z_hbm, acc.at[pl.ds(s * rpt, rpt)])

    nb = NB_LO + jnp.where(wid < NB_REM, 1, 0)
    start = NB_LO * wid + jnp.minimum(wid, NB_REM)

    pltpu.sync_copy(e3_hbm.at[0, pl.ds(start, NB_LO)],
                    rowv.at[pl.ds(0, NB_LO)])
    pltpu.sync_copy(e3_hbm.at[1, pl.ds(start, NB_LO)],
                    colv.at[pl.ds(0, NB_LO)])

    @pl.when(wid < NB_REM)
    def _():
        pltpu.sync_copy(e3_hbm.at[0, pl.ds(start + NB_LO, 1)],
                        rowv.at[pl.ds(NB_LO, 1)])
        pltpu.sync_copy(e3_hbm.at[1, pl.ds(start + NB_LO, 1)],
                        colv.at[pl.ds(NB_LO, 1)])

    plsc.subcore_barrier()

    def body(j, carry):
        pltpu.async_copy(h_hbm.at[colv.at[j]], rbuf, sem).wait()
        pltpu.sync_copy(rbuf, acc.at[rowv.at[j]], add=True)
        return carry

    lax.fori_loop(0, nb, body, 0)
    plsc.subcore_barrier()

    pltpu.sync_copy(acc.at[pl.ds(s * rpt, rpt)],
                    out_hbm.at[c, pl.ds(s * rpt, rpt)])


_sc_spmm = functools.partial(
    pl.kernel,
    out_type=jax.ShapeDtypeStruct((NC, NP, HIDDEN), jnp.float32),
    mesh=plsc.VectorSubcoreMesh(
        core_axis_name="c", subcore_axis_name="s",
        num_cores=NC, num_subcores=NS),
    scratch_types=[
        pltpu.VMEM((NB_LO + 1, BATCH), jnp.int32),
        pltpu.VMEM((NB_LO + 1, BATCH), jnp.int32),
        pltpu.VMEM((BATCH, HIDDEN), jnp.float32),
        pltpu.VMEM_SHARED((NP, HIDDEN), jnp.float32),
        pltpu.SemaphoreType.DMA,
    ],
    compiler_params=pltpu.CompilerParams(use_tc_tiling_on_sc=False),
)(_sc_spmm_body)


def _post_body(p0_ref, p1_ref, wc_ref, ws_ref, wo_ref, o_ref):
    a = p0_ref[0] + p1_ref[0]
    c1 = jnp.cos(a)
    s1 = jnp.sin(a)
    ck, sk = c1, s1
    y = lax.dot_general(c1, wc_ref[0], (((1,), (0,)), ((), ())),
                        preferred_element_type=jnp.float32)
    y = y + lax.dot_general(s1, ws_ref[0], (((1,), (0,)), ((), ())),
                            preferred_element_type=jnp.float32)
    for g in range(1, GRID):
        ck, sk = ck * c1 - sk * s1, sk * c1 + ck * s1
        y = y + lax.dot_general(ck, wc_ref[g], (((1,), (0,)), ((), ())),
                                preferred_element_type=jnp.float32)
        y = y + lax.dot_general(sk, ws_ref[g], (((1,), (0,)), ((), ())),
                                preferred_element_type=jnp.float32)
    o = lax.dot_general(y, wo_ref[...], (((1,), (1,)), ((), ())),
                        preferred_element_type=jnp.float32)
    m = jnp.max(o, axis=-1, keepdims=True)
    ex = jnp.exp(o - m)
    o_ref[...] = (o - m) - jnp.log(jnp.sum(ex, axis=-1, keepdims=True))


def _post(partials, Wc, Ws, W_out):
    return pl.pallas_call(
        _post_body,
        grid=(N // ROW_BLK,),
        in_specs=[
            pl.BlockSpec((1, ROW_BLK, HIDDEN), lambda i: (0, i, 0)),
            pl.BlockSpec((1, ROW_BLK, HIDDEN), lambda i: (1, i, 0)),
            pl.BlockSpec((GRID, HIDDEN, HIDDEN), lambda i: (0, 0, 0)),
            pl.BlockSpec((GRID, HIDDEN, HIDDEN), lambda i: (0, 0, 0)),
            pl.BlockSpec((OUT_FEAT, HIDDEN), lambda i: (0, 0)),
        ],
        out_specs=pl.BlockSpec((ROW_BLK, OUT_FEAT), lambda i: (i, 0)),
        out_shape=jax.ShapeDtypeStruct((N, OUT_FEAT), jnp.float32),
    )(partials, partials, Wc, Ws, W_out)


def kernel(x, edge_index, W_in, b_in, coeffs0, W_out):
    h = _lin_in(x, W_in, b_in)

    e3 = edge_index.reshape(2, NBAT, BATCH)
    zeros = jnp.zeros((NP // NS, HIDDEN), jnp.float32)

    partials = _sc_spmm(e3, h, zeros)

    Wc = jnp.transpose(coeffs0[0], (2, 1, 0))
    Ws = jnp.transpose(coeffs0[1], (2, 1, 0))

    return _post(partials, Wc, Ws, W_out)


# speedup vs baseline: 2.2811x; 1.0779x over previous
import functools

import jax
import jax.numpy as jnp
from jax import lax
from jax.experimental import pallas as pl
from jax.experimental.pallas import tpu as pltpu
from jax.experimental.pallas import tpu_sc as plsc

N = 10000
E = 320000
IN_FEAT = 128
HIDDEN = 64
OUT_FEAT = 64
GRID = 4

NC = 2
NS = 16
NTILES = NC * NS
BATCH = 256
NBAT = E // BATCH
NB_LO = NBAT // NTILES
NB_REM = NBAT - NB_LO * NTILES
RPT = N // NS

ROW_BLK = 2000
NBLK = N // ROW_BLK


def _lin_in_body(x_ref, w_ref, b_ref, o_ref):
    acc = lax.dot_general(
        x_ref[...], w_ref[...], (((1,), (1,)), ((), ())),
        preferred_element_type=jnp.float32)
    o_ref[...] = acc + b_ref[...]


def _lin_in(x, W_in, b_in):
    return pl.pallas_call(
        _lin_in_body,
        grid=(NBLK,),
        in_specs=[
            pl.BlockSpec((ROW_BLK, IN_FEAT), lambda i: (i, 0)),
            pl.BlockSpec((HIDDEN, IN_FEAT), lambda i: (0, 0)),
            pl.BlockSpec((1, HIDDEN), lambda i: (0, 0)),
        ],
        out_specs=pl.BlockSpec((ROW_BLK, HIDDEN), lambda i: (i, 0)),
        out_shape=jax.ShapeDtypeStruct((N, HIDDEN), jnp.float32),
    )(x, W_in, b_in.reshape(1, HIDDEN))


def _sc_spmm_body(e3_hbm, h_hbm, z_hbm, out_hbm,
                  colv, rowv, rbuf, acc, sem):
    c = lax.axis_index("c")
    s = lax.axis_index("s")
    wid = c * NS + s

    pltpu.sync_copy(z_hbm, acc.at[pl.ds(s * RPT, RPT)])

    nb = NB_LO + jnp.where(wid < NB_REM, 1, 0)
    start = NB_LO * wid + jnp.minimum(wid, NB_REM)

    pltpu.sync_copy(e3_hbm.at[0, pl.ds(start, NB_LO)],
                    rowv.at[pl.ds(0, NB_LO)])
    pltpu.sync_copy(e3_hbm.at[1, pl.ds(start, NB_LO)],
                    colv.at[pl.ds(0, NB_LO)])

    @pl.when(wid < NB_REM)
    def _():
        pltpu.sync_copy(e3_hbm.at[0, pl.ds(start + NB_LO, 1)],
                        rowv.at[pl.ds(NB_LO, 1)])
        pltpu.sync_copy(e3_hbm.at[1, pl.ds(start + NB_LO, 1)],
                        colv.at[pl.ds(NB_LO, 1)])

    plsc.subcore_barrier()

    def body(j, carry):
        pltpu.async_copy(h_hbm.at[colv.at[j]], rbuf, sem).wait()
        pltpu.sync_copy(rbuf, acc.at[rowv.at[j]], add=True)
        return carry

    lax.fori_loop(0, nb, body, 0)
    plsc.subcore_barrier()

    pltpu.sync_copy(acc.at[pl.ds(s * RPT, RPT)],
                    out_hbm.at[pl.ds(c * N + s * RPT, RPT)])


_sc_spmm = functools.partial(
    pl.kernel,
    out_type=jax.ShapeDtypeStruct((2 * N, HIDDEN), jnp.float32),
    mesh=plsc.VectorSubcoreMesh(
        core_axis_name="c", subcore_axis_name="s",
        num_cores=NC, num_subcores=NS),
    scratch_types=[
        pltpu.VMEM((NB_LO + 1, BATCH), jnp.int32),
        pltpu.VMEM((NB_LO + 1, BATCH), jnp.int32),
        pltpu.VMEM((BATCH, HIDDEN), jnp.float32),
        pltpu.VMEM_SHARED((N, HIDDEN), jnp.float32),
        pltpu.SemaphoreType.DMA,
    ],
    compiler_params=pltpu.CompilerParams(use_tc_tiling_on_sc=False),
)(_sc_spmm_body)


def _post_body(p0_ref, p1_ref, co_ref, wo_ref, o_ref):
    a = p0_ref[...] + p1_ref[...]
    c1 = jnp.cos(a)
    s1 = jnp.sin(a)
    ck, sk = c1, s1
    y = lax.dot_general(c1, co_ref[0, :, :, 0], (((1,), (1,)), ((), ())),
                        preferred_element_type=jnp.float32)
    y = y + lax.dot_general(s1, co_ref[1, :, :, 0], (((1,), (1,)), ((), ())),
                            preferred_element_type=jnp.float32)
    for g in range(1, GRID):
        ck, sk = ck * c1 - sk * s1, sk * c1 + ck * s1
        y = y + lax.dot_general(ck, co_ref[0, :, :, g],
                                (((1,), (1,)), ((), ())),
                                preferred_element_type=jnp.float32)
        y = y + lax.dot_general(sk, co_ref[1, :, :, g],
                                (((1,), (1,)), ((), ())),
                                preferred_element_type=jnp.float32)
    o = lax.dot_general(y, wo_ref[...], (((1,), (1,)), ((), ())),
                        preferred_element_type=jnp.float32)
    m = jnp.max(o, axis=-1, keepdims=True)
    ex = jnp.exp(o - m)
    o_ref[...] = (o - m) - jnp.log(jnp.sum(ex, axis=-1, keepdims=True))


def _post(partials, coeffs0, W_out):
    return pl.pallas_call(
        _post_body,
        grid=(NBLK,),
        in_specs=[
            pl.BlockSpec((ROW_BLK, HIDDEN), lambda i: (i, 0)),
            pl.BlockSpec((ROW_BLK, HIDDEN), lambda i: (NBLK + i, 0)),
            pl.BlockSpec((2, HIDDEN, HIDDEN, GRID), lambda i: (0, 0, 0, 0)),
            pl.BlockSpec((OUT_FEAT, HIDDEN), lambda i: (0, 0)),
        ],
        out_specs=pl.BlockSpec((ROW_BLK, OUT_FEAT), lambda i: (i, 0)),
        out_shape=jax.ShapeDtypeStruct((N, OUT_FEAT), jnp.float32),
    )(partials, partials, coeffs0, W_out)


def kernel(x, edge_index, W_in, b_in, coeffs0, W_out):
    h = _lin_in(x, W_in, b_in)
    e3 = edge_index.reshape(2, NBAT, BATCH)
    zeros = jnp.zeros((RPT, HIDDEN), jnp.float32)
    partials = _sc_spmm(e3, h, zeros)
    return _post(partials, coeffs0, W_out)


# speedup vs baseline: 2.4194x; 1.0606x over previous
import functools

import jax
import jax.numpy as jnp
from jax import lax
from jax.experimental import pallas as pl
from jax.experimental.pallas import tpu as pltpu
from jax.experimental.pallas import tpu_sc as plsc

N = 10000
E = 320000
IN_FEAT = 128
HIDDEN = 64
OUT_FEAT = 64
GRID = 4

NC = 2
NS = 16
NTILES = NC * NS
BATCH = 512
NBAT = E // BATCH
NB_LO = NBAT // NTILES
NB_REM = NBAT - NB_LO * NTILES
RPT = N // NS

ROW_BLK = 2000
NBLK = N // ROW_BLK


def _lin_in_body(x_ref, w_ref, b_ref, o_ref):
    acc = lax.dot_general(
        x_ref[...], w_ref[...], (((1,), (1,)), ((), ())),
        preferred_element_type=jnp.float32)
    o_ref[...] = acc + b_ref[...]


def _lin_in(x, W_in, b_in):
    return pl.pallas_call(
        _lin_in_body,
        grid=(NBLK,),
        in_specs=[
            pl.BlockSpec((ROW_BLK, IN_FEAT), lambda i: (i, 0)),
            pl.BlockSpec((HIDDEN, IN_FEAT), lambda i: (0, 0)),
            pl.BlockSpec((1, HIDDEN), lambda i: (0, 0)),
        ],
        out_specs=pl.BlockSpec((ROW_BLK, HIDDEN), lambda i: (i, 0)),
        out_shape=jax.ShapeDtypeStruct((N, HIDDEN), jnp.float32),
    )(x, W_in, b_in.reshape(1, HIDDEN))


def _sc_spmm_body(e3_hbm, h_hbm, z_hbm, out_hbm,
                  colv, rowv, rbuf, acc, sem):
    c = lax.axis_index("c")
    s = lax.axis_index("s")
    wid = c * NS + s

    pltpu.sync_copy(z_hbm, acc.at[pl.ds(s * RPT, RPT)])

    nb = NB_LO + jnp.where(wid < NB_REM, 1, 0)
    start = NB_LO * wid + jnp.minimum(wid, NB_REM)

    pltpu.sync_copy(e3_hbm.at[0, pl.ds(start, NB_LO)],
                    rowv.at[pl.ds(0, NB_LO)])
    pltpu.sync_copy(e3_hbm.at[1, pl.ds(start, NB_LO)],
                    colv.at[pl.ds(0, NB_LO)])

    @pl.when(wid < NB_REM)
    def _():
        pltpu.sync_copy(e3_hbm.at[0, pl.ds(start + NB_LO, 1)],
                        rowv.at[pl.ds(NB_LO, 1)])
        pltpu.sync_copy(e3_hbm.at[1, pl.ds(start + NB_LO, 1)],
                        colv.at[pl.ds(NB_LO, 1)])

    plsc.subcore_barrier()

    def body(j, carry):
        pltpu.async_copy(h_hbm.at[colv.at[j]], rbuf, sem).wait()
        pltpu.sync_copy(rbuf, acc.at[rowv.at[j]], add=True)
        return carry

    lax.fori_loop(0, nb, body, 0)
    plsc.subcore_barrier()

    pltpu.sync_copy(acc.at[pl.ds(s * RPT, RPT)],
                    out_hbm.at[pl.ds(c * N + s * RPT, RPT)])


_sc_spmm = functools.partial(
    pl.kernel,
    out_type=jax.ShapeDtypeStruct((2 * N, HIDDEN), jnp.float32),
    mesh=plsc.VectorSubcoreMesh(
        core_axis_name="c", subcore_axis_name="s",
        num_cores=NC, num_subcores=NS),
    scratch_types=[
        pltpu.VMEM((NB_LO + 1, BATCH), jnp.int32),
        pltpu.VMEM((NB_LO + 1, BATCH), jnp.int32),
        pltpu.VMEM((BATCH, HIDDEN), jnp.float32),
        pltpu.VMEM_SHARED((N, HIDDEN), jnp.float32),
        pltpu.SemaphoreType.DMA,
    ],
    compiler_params=pltpu.CompilerParams(use_tc_tiling_on_sc=False),
)(_sc_spmm_body)


def _post_body(p0_ref, p1_ref, co_ref, wo_ref, o_ref):
    a = p0_ref[...] + p1_ref[...]
    c1 = jnp.cos(a)
    s1 = jnp.sin(a)
    ck, sk = c1, s1
    y = lax.dot_general(c1, co_ref[0, :, :, 0], (((1,), (1,)), ((), ())),
                        preferred_element_type=jnp.float32)
    y = y + lax.dot_general(s1, co_ref[1, :, :, 0], (((1,), (1,)), ((), ())),
                            preferred_element_type=jnp.float32)
    for g in range(1, GRID):
        ck, sk = ck * c1 - sk * s1, sk * c1 + ck * s1
        y = y + lax.dot_general(ck, co_ref[0, :, :, g],
                                (((1,), (1,)), ((), ())),
                                preferred_element_type=jnp.float32)
        y = y + lax.dot_general(sk, co_ref[1, :, :, g],
                                (((1,), (1,)), ((), ())),
                                preferred_element_type=jnp.float32)
    o = lax.dot_general(y, wo_ref[...], (((1,), (1,)), ((), ())),
                        preferred_element_type=jnp.float32)
    m = jnp.max(o, axis=-1, keepdims=True)
    ex = jnp.exp(o - m)
    o_ref[...] = (o - m) - jnp.log(jnp.sum(ex, axis=-1, keepdims=True))


def _post(partials, coeffs0, W_out):
    return pl.pallas_call(
        _post_body,
        grid=(NBLK,),
        in_specs=[
            pl.BlockSpec((ROW_BLK, HIDDEN), lambda i: (i, 0)),
            pl.BlockSpec((ROW_BLK, HIDDEN), lambda i: (NBLK + i, 0)),
            pl.BlockSpec((2, HIDDEN, HIDDEN, GRID), lambda i: (0, 0, 0, 0)),
            pl.BlockSpec((OUT_FEAT, HIDDEN), lambda i: (0, 0)),
        ],
        out_specs=pl.BlockSpec((ROW_BLK, OUT_FEAT), lambda i: (i, 0)),
        out_shape=jax.ShapeDtypeStruct((N, OUT_FEAT), jnp.float32),
    )(partials, partials, coeffs0, W_out)


def kernel(x, edge_index, W_in, b_in, coeffs0, W_out):
    h = _lin_in(x, W_in, b_in)
    e3 = edge_index.reshape(2, NBAT, BATCH)
    zeros = jnp.zeros((RPT, HIDDEN), jnp.float32)
    partials = _sc_spmm(e3, h, zeros)
    return _post(partials, coeffs0, W_out)


# speedup vs baseline: 2.5013x; 1.0339x over previous
import functools

import jax
import jax.numpy as jnp
from jax import lax
from jax.experimental import pallas as pl
from jax.experimental.pallas import tpu as pltpu
from jax.experimental.pallas import tpu_sc as plsc

N = 10000
E = 320000
IN_FEAT = 128
HIDDEN = 64
OUT_FEAT = 64
GRID = 4

NC = 2
NS = 16
NTILES = NC * NS
BATCH = 1000
NBAT = E // BATCH
NB_LO = NBAT // NTILES
NB_REM = NBAT - NB_LO * NTILES
RPT = N // NS

ROW_BLK = 2000
NBLK = N // ROW_BLK


def _lin_in_body(x_ref, w_ref, b_ref, o_ref):
    acc = lax.dot_general(
        x_ref[...], w_ref[...], (((1,), (1,)), ((), ())),
        preferred_element_type=jnp.float32)
    o_ref[...] = acc + b_ref[...]


def _lin_in(x, W_in, b_in):
    return pl.pallas_call(
        _lin_in_body,
        grid=(NBLK,),
        in_specs=[
            pl.BlockSpec((ROW_BLK, IN_FEAT), lambda i: (i, 0)),
            pl.BlockSpec((HIDDEN, IN_FEAT), lambda i: (0, 0)),
            pl.BlockSpec((1, HIDDEN), lambda i: (0, 0)),
        ],
        out_specs=pl.BlockSpec((ROW_BLK, HIDDEN), lambda i: (i, 0)),
        out_shape=jax.ShapeDtypeStruct((N, HIDDEN), jnp.float32),
    )(x, W_in, b_in.reshape(1, HIDDEN))


def _sc_spmm_body(e3_hbm, h_hbm, z_hbm, out_hbm,
                  colv, rowv, rbuf, acc, sem):
    c = lax.axis_index("c")
    s = lax.axis_index("s")
    wid = c * NS + s

    pltpu.sync_copy(z_hbm, acc.at[pl.ds(s * RPT, RPT)])

    nb = NB_LO + jnp.where(wid < NB_REM, 1, 0)
    start = NB_LO * wid + jnp.minimum(wid, NB_REM)

    pltpu.sync_copy(e3_hbm.at[0, pl.ds(start, NB_LO)],
                    rowv.at[pl.ds(0, NB_LO)])
    pltpu.sync_copy(e3_hbm.at[1, pl.ds(start, NB_LO)],
                    colv.at[pl.ds(0, NB_LO)])

    @pl.when(wid < NB_REM)
    def _():
        pltpu.sync_copy(e3_hbm.at[0, pl.ds(start + NB_LO, 1)],
                        rowv.at[pl.ds(NB_LO, 1)])
        pltpu.sync_copy(e3_hbm.at[1, pl.ds(start + NB_LO, 1)],
                        colv.at[pl.ds(NB_LO, 1)])

    plsc.subcore_barrier()

    def body(j, carry):
        pltpu.async_copy(h_hbm.at[colv.at[j]], rbuf, sem).wait()
        pltpu.sync_copy(rbuf, acc.at[rowv.at[j]], add=True)
        return carry

    lax.fori_loop(0, nb, body, 0)
    plsc.subcore_barrier()

    pltpu.sync_copy(acc.at[pl.ds(s * RPT, RPT)],
                    out_hbm.at[pl.ds(c * N + s * RPT, RPT)])


_sc_spmm = functools.partial(
    pl.kernel,
    out_type=jax.ShapeDtypeStruct((2 * N, HIDDEN), jnp.float32),
    mesh=plsc.VectorSubcoreMesh(
        core_axis_name="c", subcore_axis_name="s",
        num_cores=NC, num_subcores=NS),
    scratch_types=[
        pltpu.VMEM((NB_LO + 1, BATCH), jnp.int32),
        pltpu.VMEM((NB_LO + 1, BATCH), jnp.int32),
        pltpu.VMEM((BATCH, HIDDEN), jnp.float32),
        pltpu.VMEM_SHARED((N, HIDDEN), jnp.float32),
        pltpu.SemaphoreType.DMA,
    ],
    compiler_params=pltpu.CompilerParams(use_tc_tiling_on_sc=False),
)(_sc_spmm_body)


def _post_body(p0_ref, p1_ref, co_ref, wo_ref, o_ref):
    a = p0_ref[...] + p1_ref[...]
    c1 = jnp.cos(a)
    s1 = jnp.sin(a)
    ck, sk = c1, s1
    y = lax.dot_general(c1, co_ref[0, :, :, 0], (((1,), (1,)), ((), ())),
                        preferred_element_type=jnp.float32)
    y = y + lax.dot_general(s1, co_ref[1, :, :, 0], (((1,), (1,)), ((), ())),
                            preferred_element_type=jnp.float32)
    for g in range(1, GRID):
        ck, sk = ck * c1 - sk * s1, sk * c1 + ck * s1
        y = y + lax.dot_general(ck, co_ref[0, :, :, g],
                                (((1,), (1,)), ((), ())),
                                preferred_element_type=jnp.float32)
        y = y + lax.dot_general(sk, co_ref[1, :, :, g],
                                (((1,), (1,)), ((), ())),
                                preferred_element_type=jnp.float32)
    o = lax.dot_general(y, wo_ref[...], (((1,), (1,)), ((), ())),
                        preferred_element_type=jnp.float32)
    m = jnp.max(o, axis=-1, keepdims=True)
    ex = jnp.exp(o - m)
    o_ref[...] = (o - m) - jnp.log(jnp.sum(ex, axis=-1, keepdims=True))


def _post(partials, coeffs0, W_out):
    return pl.pallas_call(
        _post_body,
        grid=(NBLK,),
        in_specs=[
            pl.BlockSpec((ROW_BLK, HIDDEN), lambda i: (i, 0)),
            pl.BlockSpec((ROW_BLK, HIDDEN), lambda i: (NBLK + i, 0)),
            pl.BlockSpec((2, HIDDEN, HIDDEN, GRID), lambda i: (0, 0, 0, 0)),
            pl.BlockSpec((OUT_FEAT, HIDDEN), lambda i: (0, 0)),
        ],
        out_specs=pl.BlockSpec((ROW_BLK, OUT_FEAT), lambda i: (i, 0)),
        out_shape=jax.ShapeDtypeStruct((N, OUT_FEAT), jnp.float32),
    )(partials, partials, coeffs0, W_out)


def kernel(x, edge_index, W_in, b_in, coeffs0, W_out):
    h = _lin_in(x, W_in, b_in)
    e3 = edge_index.reshape(2, NBAT, BATCH)
    zeros = jnp.zeros((RPT, HIDDEN), jnp.float32)
    partials = _sc_spmm(e3, h, zeros)
    return _post(partials, coeffs0, W_out)


# speedup vs baseline: 2.6953x; 1.0776x over previous
import functools

import jax
import jax.numpy as jnp
from jax import lax
from jax.experimental import pallas as pl
from jax.experimental.pallas import tpu as pltpu
from jax.experimental.pallas import tpu_sc as plsc

N = 10000
E = 320000
IN_FEAT = 128
HIDDEN = 64
OUT_FEAT = 64
GRID = 4

NC = 2
NS = 16
NTILES = NC * NS
BATCH = 1000
NBAT = E // BATCH
NB_LO = NBAT // NTILES
NB_REM = NBAT - NB_LO * NTILES
RPT = N // NS

ROW_BLK = 2000
NBLK = N // ROW_BLK


def _lin_in_body(x_ref, w_ref, b_ref, o_ref):
    acc = lax.dot_general(
        x_ref[...], w_ref[...], (((1,), (1,)), ((), ())),
        preferred_element_type=jnp.float32)
    o_ref[...] = acc + b_ref[...]


def _lin_in(x, W_in, b_in):
    return pl.pallas_call(
        _lin_in_body,
        grid=(NBLK,),
        in_specs=[
            pl.BlockSpec((ROW_BLK, IN_FEAT), lambda i: (i, 0)),
            pl.BlockSpec((HIDDEN, IN_FEAT), lambda i: (0, 0)),
            pl.BlockSpec((1, HIDDEN), lambda i: (0, 0)),
        ],
        out_specs=pl.BlockSpec((ROW_BLK, HIDDEN), lambda i: (i, 0)),
        out_shape=jax.ShapeDtypeStruct((N, HIDDEN), jnp.float32),
    )(x, W_in, b_in.reshape(1, HIDDEN))


def _sc_spmm_body(e3_hbm, h_hbm, z_hbm, out_hbm,
                  colv, rowv, rbuf, acc, sem):
    c = lax.axis_index("c")
    s = lax.axis_index("s")
    wid = c * NS + s

    pltpu.sync_copy(z_hbm, acc.at[pl.ds(s * RPT, RPT)])

    nb = NB_LO + jnp.where(wid < NB_REM, 1, 0)
    start = NB_LO * wid + jnp.minimum(wid, NB_REM)

    pltpu.sync_copy(e3_hbm.at[0, pl.ds(start, NB_LO)],
                    rowv.at[pl.ds(0, NB_LO)])
    pltpu.sync_copy(e3_hbm.at[1, pl.ds(start, NB_LO)],
                    colv.at[pl.ds(0, NB_LO)])

    @pl.when(wid < NB_REM)
    def _():
        pltpu.sync_copy(e3_hbm.at[0, pl.ds(start + NB_LO, 1)],
                        rowv.at[pl.ds(NB_LO, 1)])
        pltpu.sync_copy(e3_hbm.at[1, pl.ds(start + NB_LO, 1)],
                        colv.at[pl.ds(NB_LO, 1)])

    plsc.subcore_barrier()

    def body(j, carry):
        pltpu.async_copy(h_hbm.at[colv.at[j]], rbuf, sem).wait()
        pltpu.sync_copy(rbuf, acc.at[rowv.at[j]], add=True)
        return carry

    lax.fori_loop(0, nb, body, 0)
    plsc.subcore_barrier()

    pltpu.sync_copy(acc.at[pl.ds(s * RPT, RPT)],
                    out_hbm.at[pl.ds(c * N + s * RPT, RPT)])


_sc_spmm = functools.partial(
    pl.kernel,
    out_type=jax.ShapeDtypeStruct((2 * N, HIDDEN), jnp.float32),
    mesh=plsc.VectorSubcoreMesh(
        core_axis_name="c", subcore_axis_name="s",
        num_cores=NC, num_subcores=NS),
    scratch_types=[
        pltpu.VMEM((NB_LO + 1, BATCH), jnp.int32),
        pltpu.VMEM((NB_LO + 1, BATCH), jnp.int32),
        pltpu.VMEM((BATCH, HIDDEN), jnp.float32),
        pltpu.VMEM_SHARED((N, HIDDEN), jnp.float32),
        pltpu.SemaphoreType.DMA,
    ],
    compiler_params=pltpu.CompilerParams(use_tc_tiling_on_sc=False),
)(_sc_spmm_body)


def _post_body(p0_ref, p1_ref, wc_ref, ws_ref, wo_ref, o_ref):
    a = p0_ref[...] + p1_ref[...]
    c1 = jnp.cos(a)
    s1 = jnp.sin(a)
    ck, sk = c1, s1
    y = lax.dot_general(c1, wc_ref[0], (((1,), (0,)), ((), ())),
                        preferred_element_type=jnp.float32)
    y = y + lax.dot_general(s1, ws_ref[0], (((1,), (0,)), ((), ())),
                            preferred_element_type=jnp.float32)
    for g in range(1, GRID):
        ck, sk = ck * c1 - sk * s1, sk * c1 + ck * s1
        y = y + lax.dot_general(ck, wc_ref[g], (((1,), (0,)), ((), ())),
                                preferred_element_type=jnp.float32)
        y = y + lax.dot_general(sk, ws_ref[g], (((1,), (0,)), ((), ())),
                                preferred_element_type=jnp.float32)
    o = lax.dot_general(y, wo_ref[...], (((1,), (1,)), ((), ())),
                        preferred_element_type=jnp.float32)
    m = jnp.max(o, axis=-1, keepdims=True)
    ex = jnp.exp(o - m)
    o_ref[...] = (o - m) - jnp.log(jnp.sum(ex, axis=-1, keepdims=True))


def _post(partials, Wc, Ws, W_out):
    return pl.pallas_call(
        _post_body,
        grid=(NBLK,),
        in_specs=[
            pl.BlockSpec((ROW_BLK, HIDDEN), lambda i: (i, 0)),
            pl.BlockSpec((ROW_BLK, HIDDEN), lambda i: (NBLK + i, 0)),
            pl.BlockSpec((GRID, HIDDEN, HIDDEN), lambda i: (0, 0, 0)),
            pl.BlockSpec((GRID, HIDDEN, HIDDEN), lambda i: (0, 0, 0)),
            pl.BlockSpec((OUT_FEAT, HIDDEN), lambda i: (0, 0)),
        ],
        out_specs=pl.BlockSpec((ROW_BLK, OUT_FEAT), lambda i: (i, 0)),
        out_shape=jax.ShapeDtypeStruct((N, OUT_FEAT), jnp.float32),
    )(partials, partials, Wc, Ws, W_out)


def kernel(x, edge_index, W_in, b_in, coeffs0, W_out):
    h = _lin_in(x, W_in, b_in)
    e3 = edge_index.reshape(2, NBAT, BATCH)
    zeros = jnp.zeros((RPT, HIDDEN), jnp.float32)
    partials = _sc_spmm(e3, h, zeros)
    Wc = jnp.transpose(coeffs0[0], (2, 1, 0))
    Ws = jnp.transpose(coeffs0[1], (2, 1, 0))
    return _post(partials, Wc, Ws, W_out)


# speedup vs baseline: 2.8760x; 1.0670x over previous
import functools

import jax
import jax.numpy as jnp
from jax import lax
from jax.experimental import pallas as pl
from jax.experimental.pallas import tpu as pltpu
from jax.experimental.pallas import tpu_sc as plsc

N = 10000
E = 320000
IN_FEAT = 128
HIDDEN = 64
OUT_FEAT = 64
GRID = 4

NC = 2
NS = 16
NTILES = NC * NS
BATCH = 500
NBAT = E // BATCH
NB = NBAT // NTILES
RPT = N // NS

ROW_BLK = 2000
NBLK = N // ROW_BLK


def _lin_in_body(x_ref, w_ref, b_ref, o_ref):
    acc = lax.dot_general(
        x_ref[...], w_ref[...], (((1,), (1,)), ((), ())),
        preferred_element_type=jnp.float32)
    o_ref[...] = acc + b_ref[...]


def _lin_in(x, W_in, b_in):
    return pl.pallas_call(
        _lin_in_body,
        grid=(NBLK,),
        in_specs=[
            pl.BlockSpec((ROW_BLK, IN_FEAT), lambda i: (i, 0)),
            pl.BlockSpec((HIDDEN, IN_FEAT), lambda i: (0, 0)),
            pl.BlockSpec((1, HIDDEN), lambda i: (0, 0)),
        ],
        out_specs=pl.BlockSpec((ROW_BLK, HIDDEN), lambda i: (i, 0)),
        out_shape=jax.ShapeDtypeStruct((N, HIDDEN), jnp.float32),
    )(x, W_in, b_in.reshape(1, HIDDEN))


def _sc_spmm_body(e3_hbm, h_hbm, z_hbm, out_hbm,
                  colv, rowv, rba, rbb, acc, semga, semgb):
    c = lax.axis_index("c")
    s = lax.axis_index("s")
    wid = c * NS + s

    pltpu.sync_copy(z_hbm, acc.at[pl.ds(s * RPT, RPT)])

    start = NB * wid
    pltpu.sync_copy(e3_hbm.at[0, pl.ds(start, NB)], rowv)
    pltpu.sync_copy(e3_hbm.at[1, pl.ds(start, NB)], colv)

    plsc.subcore_barrier()

    pltpu.async_copy(h_hbm.at[colv.at[0]], rba, semga)

    def body(j2, carry):
        j = j2 * 2
        pltpu.make_async_copy(h_hbm.at[colv.at[j]], rba, semga).wait()
        pltpu.async_copy(h_hbm.at[colv.at[j + 1]], rbb, semgb)
        pltpu.sync_copy(rba, acc.at[rowv.at[j]], add=True)
        pltpu.make_async_copy(h_hbm.at[colv.at[j + 1]], rbb, semgb).wait()
        pltpu.async_copy(h_hbm.at[colv.at[j + 2]], rba, semga)
        pltpu.sync_copy(rbb, acc.at[rowv.at[j + 1]], add=True)
        return carry

    lax.fori_loop(0, NB // 2 - 1, body, 0)
    pltpu.make_async_copy(h_hbm.at[colv.at[NB - 2]], rba, semga).wait()
    pltpu.async_copy(h_hbm.at[colv.at[NB - 1]], rbb, semgb)
    pltpu.sync_copy(rba, acc.at[rowv.at[NB - 2]], add=True)
    pltpu.make_async_copy(h_hbm.at[colv.at[NB - 1]], rbb, semgb).wait()
    pltpu.sync_copy(rbb, acc.at[rowv.at[NB - 1]], add=True)
    plsc.subcore_barrier()

    pltpu.sync_copy(acc.at[pl.ds(s * RPT, RPT)],
                    out_hbm.at[pl.ds(c * N + s * RPT, RPT)])


_sc_spmm = functools.partial(
    pl.kernel,
    out_type=jax.ShapeDtypeStruct((2 * N, HIDDEN), jnp.float32),
    mesh=plsc.VectorSubcoreMesh(
        core_axis_name="c", subcore_axis_name="s",
        num_cores=NC, num_subcores=NS),
    scratch_types=[
        pltpu.VMEM((NB, BATCH), jnp.int32),
        pltpu.VMEM((NB, BATCH), jnp.int32),
        pltpu.VMEM((BATCH, HIDDEN), jnp.float32),
        pltpu.VMEM((BATCH, HIDDEN), jnp.float32),
        pltpu.VMEM_SHARED((N, HIDDEN), jnp.float32),
        pltpu.SemaphoreType.DMA,
        pltpu.SemaphoreType.DMA,
    ],
    compiler_params=pltpu.CompilerParams(use_tc_tiling_on_sc=False),
)(_sc_spmm_body)


def _post_body(p0_ref, p1_ref, wc_ref, ws_ref, wo_ref, o_ref):
    a = p0_ref[...] + p1_ref[...]
    c1 = jnp.cos(a)
    s1 = jnp.sin(a)
    ck, sk = c1, s1
    y = lax.dot_general(c1, wc_ref[0], (((1,), (0,)), ((), ())),
                        preferred_element_type=jnp.float32)
    y = y + lax.dot_general(s1, ws_ref[0], (((1,), (0,)), ((), ())),
                            preferred_element_type=jnp.float32)
    for g in range(1, GRID):
        ck, sk = ck * c1 - sk * s1, sk * c1 + ck * s1
        y = y + lax.dot_general(ck, wc_ref[g], (((1,), (0,)), ((), ())),
                                preferred_element_type=jnp.float32)
        y = y + lax.dot_general(sk, ws_ref[g], (((1,), (0,)), ((), ())),
                                preferred_element_type=jnp.float32)
    o = lax.dot_general(y, wo_ref[...], (((1,), (1,)), ((), ())),
                        preferred_element_type=jnp.float32)
    m = jnp.max(o, axis=-1, keepdims=True)
    ex = jnp.exp(o - m)
    o_ref[...] = (o - m) - jnp.log(jnp.sum(ex, axis=-1, keepdims=True))


def _post(partials, Wc, Ws, W_out):
    return pl.pallas_call(
        _post_body,
        grid=(NBLK,),
        in_specs=[
            pl.BlockSpec((ROW_BLK, HIDDEN), lambda i: (i, 0)),
            pl.BlockSpec((ROW_BLK, HIDDEN), lambda i: (NBLK + i, 0)),
            pl.BlockSpec((GRID, HIDDEN, HIDDEN), lambda i: (0, 0, 0)),
            pl.BlockSpec((GRID, HIDDEN, HIDDEN), lambda i: (0, 0, 0)),
            pl.BlockSpec((OUT_FEAT, HIDDEN), lambda i: (0, 0)),
        ],
        out_specs=pl.BlockSpec((ROW_BLK, OUT_FEAT), lambda i: (i, 0)),
        out_shape=jax.ShapeDtypeStruct((N, OUT_FEAT), jnp.float32),
    )(partials, partials, Wc, Ws, W_out)


def kernel(x, edge_index, W_in, b_in, coeffs0, W_out):
    h = _lin_in(x, W_in, b_in)
    e3 = edge_index.reshape(2, NBAT, BATCH)
    zeros = jnp.zeros((RPT, HIDDEN), jnp.float32)
    partials = _sc_spmm(e3, h, zeros)
    Wc = jnp.transpose(coeffs0[0], (2, 1, 0))
    Ws = jnp.transpose(coeffs0[1], (2, 1, 0))
    return _post(partials, Wc, Ws, W_out)
